# Initial kernel scaffold; baseline (speedup 1.0000x reference)
import jax
import jax.numpy as jnp
from jax.experimental import pallas as pl


def kernel(x, edge_index, params):
    raise NotImplementedError("write your pallas kernel here")


# speedup vs baseline: 5.4798x; 5.4798x over previous
import functools

import jax
import jax.numpy as jnp
from jax import lax
from jax.experimental import pallas as pl
from jax.experimental.pallas import tpu as pltpu
from jax.experimental.pallas import tpu_sc as plsc

_N = 10000
_H = 200
_E = 320000
_ROWS = 2000
_NW = 32
_EPW = _E // _NW
_NPB = 64
_NRND = 5
_NOUT = _NW * _NRND * _NPB
_GB = 128
_G = 48


def _wid():
    return lax.axis_index("c") * 16 + lax.axis_index("s")


def _gather_body(cp, h_hbm, src_hbm, msgs_hbm, src_v, buf0, buf1, gsem, wsem):
    w = _wid()
    ebase = w * _EPW
    pltpu.sync_copy(src_hbm.at[pl.ds(ebase, _EPW)], src_v)
    bufs = (buf0, buf1)
    nfull = _EPW // _GB
    rem = _EPW - nfull * _GB

    def wdesc(b, buf, n):
        return pltpu.make_async_copy(
            buf.at[pl.ds(0, n)], msgs_hbm.at[pl.ds(ebase + b * _GB, n)], wsem)

    for b in range(nfull + 1):
        n = _GB if b < nfull else rem
        buf = bufs[b % 2]
        if b >= 2:
            wdesc(b - 2, buf, _GB).wait()
        g = pltpu.make_async_copy(
            h_hbm.at[src_v.at[pl.ds(b * _GB, n)]], buf.at[pl.ds(0, n)], gsem)
        g.start()
        g.wait()
        wdesc(b, buf, n).start()
    wdesc(nfull - 1, bufs[(nfull - 1) % 2], _GB).wait()
    wdesc(nfull, bufs[nfull % 2], rem).wait()


@functools.cache
def _make_gather(cp):
    mesh = plsc.VectorSubcoreMesh(core_axis_name="c", subcore_axis_name="s")
    f32, i32 = jnp.float32, jnp.int32
    return pl.kernel(
        functools.partial(_gather_body, cp),
        out_type=[jax.ShapeDtypeStruct((_E + 96, cp), f32)],
        mesh=mesh,
        scratch_types=[
            pltpu.VMEM((_EPW,), i32),
            pltpu.VMEM((_GB, cp), f32),
            pltpu.VMEM((_GB, cp), f32),
            pltpu.SemaphoreType.DMA,
            pltpu.SemaphoreType.DMA,
        ],
    )


def _accum_body(cpt, cp, msgs_hbm, dst_hbm, offs_hbm,
                s_hbm, s2_hbm, mn_hbm, mx_hbm, cnt_hbm,
                offs_v, dstb0, dstb1, sb0, sb1,
                acc_s, acc_s2, acc_mn, acc_mx, acc_cnt,
                dsem0, dsem1, ssem0, ssem1):
    nvu = cp // 16
    w = _wid()
    iota = lax.broadcasted_iota(jnp.int32, (16,), 0)
    zerov = jnp.zeros((16,), jnp.float32)
    onehot = jnp.where(iota == 0, 1.0, 0.0).astype(jnp.float32)
    infv = jnp.full((16,), jnp.inf, jnp.float32)

    pltpu.sync_copy(offs_hbm, offs_v)

    dstbs = (dstb0, dstb1)
    sbs = (sb0, sb1)
    dsems = (dsem0, dsem1)
    ssems = (ssem0, ssem1)

    def round_body(r, _carry):
        blk = w * _NRND + r
        base = blk * _NPB
        ov = offs_v[blk, pl.ds(0, 16)]
        e0 = ov[0]
        e1 = ov[1]
        eb = (e0 // 8) * 8
        lo = e0 - eb
        ne = e1 - eb
        nchunk = (ne + _G - 1) // _G
        npair = (nchunk + 1) // 2

        def zrow(rr, _):
            for j in range(nvu):
                sl = pl.ds(j * 16, 16)
                acc_s[rr, sl] = zerov
                acc_s2[rr, sl] = zerov
                acc_mn[rr, sl] = infv
                acc_mx[rr, sl] = -infv
            acc_cnt[rr, pl.ds(0, 16)] = zerov
            return 0

        lax.fori_loop(0, _NPB + 1, zrow, 0)

        def start_chunk(c, b):
            ok = c < nchunk

            @pl.when(ok)
            def _():
                pltpu.make_async_copy(
                    dst_hbm.at[pl.ds(eb + c * _G, _G)],
                    dstbs[b].at[pl.ds(0, _G)], dsems[b]).start()
                pltpu.make_async_copy(
                    msgs_hbm.at[pl.ds(eb + c * _G, _G)],
                    sbs[b], ssems[b]).start()

        def wait_chunk(c, b):
            ok = c < nchunk

            @pl.when(ok)
            def _():
                pltpu.make_async_copy(
                    dst_hbm.at[pl.ds(0, _G)],
                    dstbs[b].at[pl.ds(0, _G)], dsems[b]).wait()
                pltpu.make_async_copy(
                    msgs_hbm.at[pl.ds(0, _G)], sbs[b], ssems[b]).wait()

        def do_chunk(c, b):
            def group(g, _):
                dvec = dstbs[b][pl.ds(g * 8, 16)]
                for lane in range(8):
                    eidx = c * _G + g * 8 + lane
                    valid = jnp.logical_and(eidx >= lo, eidx < ne)
                    d = dvec[lane]
                    nl = jnp.where(valid, d - base, _NPB)
                    ridx = g * 8 + lane
                    rows = [sbs[b][ridx, pl.ds(j * 16, 16)]
                            for j in range(nvu)]
                    for j in range(nvu):
                        sl = pl.ds(j * 16, 16)
                        plsc.addupdate(acc_s.at[nl, sl], rows[j])
                        plsc.addupdate(acc_s2.at[nl, sl], rows[j] * rows[j])
                        acc_mn[nl, sl] = jnp.minimum(acc_mn[nl, sl], rows[j])
                        acc_mx[nl, sl] = jnp.maximum(acc_mx[nl, sl], rows[j])
                    plsc.addupdate(acc_cnt.at[nl, pl.ds(0, 16)], onehot)
                return 0

            lax.fori_loop(0, _G // 8, group, 0)

        start_chunk(0, 0)
        start_chunk(1, 1)

        def pair(i, _):
            c = 2 * i
            wait_chunk(c, 0)
            do_chunk(c, 0)
            start_chunk(c + 2, 0)
            wait_chunk(c + 1, 1)
            do_chunk(c + 1, 1)
            start_chunk(c + 3, 1)
            return 0

        lax.fori_loop(0, npair, pair, 0)

        for acc, hbm in ((acc_s, s_hbm), (acc_s2, s2_hbm),
                         (acc_mn, mn_hbm), (acc_mx, mx_hbm)):
            pltpu.sync_copy(acc.at[pl.ds(0, _NPB)],
                            hbm.at[pl.ds(base, _NPB)])
        pltpu.sync_copy(acc_cnt.at[pl.ds(0, _NPB)],
                        cnt_hbm.at[pl.ds(base, _NPB)])
        return 0

    lax.fori_loop(0, _NRND, round_body, 0)


@functools.cache
def _make_accum(cpt, cp):
    mesh = plsc.VectorSubcoreMesh(core_axis_name="c", subcore_axis_name="s")
    f32, i32 = jnp.float32, jnp.int32
    return pl.kernel(
        functools.partial(_accum_body, cpt, cp),
        out_type=[
            jax.ShapeDtypeStruct((_NOUT, cp), f32),
            jax.ShapeDtypeStruct((_NOUT, cp), f32),
            jax.ShapeDtypeStruct((_NOUT, cp), f32),
            jax.ShapeDtypeStruct((_NOUT, cp), f32),
            jax.ShapeDtypeStruct((_NOUT, 16), f32),
        ],
        mesh=mesh,
        scratch_types=[
            pltpu.VMEM((_NW * _NRND, 16), i32),
            pltpu.VMEM((_G + 16,), i32),
            pltpu.VMEM((_G + 16,), i32),
            pltpu.VMEM((_G, cpt), f32),
            pltpu.VMEM((_G, cpt), f32),
            pltpu.VMEM((_NPB + 1, cp), f32),
            pltpu.VMEM((_NPB + 1, cp), f32),
            pltpu.VMEM((_NPB + 1, cp), f32),
            pltpu.VMEM((_NPB + 1, cp), f32),
            pltpu.VMEM((_NPB + 1, 16), f32),
            pltpu.SemaphoreType.DMA,
            pltpu.SemaphoreType.DMA,
            pltpu.SemaphoreType.DMA,
            pltpu.SemaphoreType.DMA,
        ],
    )


def _dense_body(C, cnt_ref, s_ref, s2_ref, mn_ref, mx_ref, h_ref,
                wp_ref, bp_ref, wl_ref, bl_ref, wr_ref, br_ref,
                out_ref, colsum_ref, colsq_ref):
    cnt = cnt_ref[...][:, 0:1]
    denom = jnp.maximum(cnt, 1.0)
    has = cnt > 0
    s = s_ref[...][:, :C]
    s2 = s2_ref[...][:, :C]
    mn = jnp.where(has, mn_ref[...][:, :C], 0.0)
    mx = jnp.where(has, mx_ref[...][:, :C], 0.0)
    h = h_ref[...][:, :C]
    mean = s / denom
    mean2 = s2 / denom
    var = mean2 - mean * mean
    std = jnp.sqrt(jnp.clip(var, 1e-5, None))

    def dot(a, b):
        return jax.lax.dot_general(
            a.astype(jnp.bfloat16), b.astype(jnp.bfloat16),
            (((1,), (0,)), ((), ())),
            preferred_element_type=jnp.float32)

    cat = jnp.concatenate([mean, mn, mx, std, var], axis=1)
    aggr = dot(cat, wp_ref[...]) + bp_ref[...]
    out = dot(aggr, wl_ref[...]) + bl_ref[...] + dot(h, wr_ref[...]) + br_ref[...]
    nrm = jnp.sqrt(jnp.sum(out * out, axis=1, keepdims=True))
    out = out / jnp.maximum(nrm, 1e-12)
    out = jnp.maximum(out, 0.0)
    out_ref[...] = out
    colsum_ref[...] = jnp.sum(out, axis=0, keepdims=True).reshape(1, 1, _H)
    colsq_ref[...] = jnp.sum(out * out, axis=0, keepdims=True).reshape(1, 1, _H)


def _dense_layer(cnt, s, s2, mn, mx, h, p, C):
    nb = _N // _ROWS
    cp = s.shape[1]
    hw = h.shape[1]
    row = lambda i: (i, 0)
    full = lambda i: (0, 0)
    rspec = lambda c: pl.BlockSpec((_ROWS, c), row)
    fspec = lambda a, b: pl.BlockSpec((a, b), full)
    return pl.pallas_call(
        functools.partial(_dense_body, C),
        grid=(nb,),
        in_specs=[
            rspec(16), rspec(cp), rspec(cp), rspec(cp), rspec(cp), rspec(hw),
            fspec(5 * C, _H), fspec(1, _H), fspec(_H, _H), fspec(1, _H),
            fspec(C, _H), fspec(1, _H),
        ],
        out_specs=[
            rspec(_H),
            pl.BlockSpec((1, 1, _H), lambda i: (i, 0, 0)),
            pl.BlockSpec((1, 1, _H), lambda i: (i, 0, 0)),
        ],
        out_shape=[
            jax.ShapeDtypeStruct((_N, _H), jnp.float32),
            jax.ShapeDtypeStruct((nb, 1, _H), jnp.float32),
            jax.ShapeDtypeStruct((nb, 1, _H), jnp.float32),
        ],
    )(cnt, s, s2, mn, mx, h,
      p["W_proj"], p["b_proj"].reshape(1, _H), p["W_l"], p["b_l"].reshape(1, _H),
      p["W_r"], p["b_r"].reshape(1, _H))


def _bn_body(h_ref, colsum_ref, colsq_ref, g_ref, b_ref, out_ref):
    cs = jnp.sum(colsum_ref[...], axis=0)
    cq = jnp.sum(colsq_ref[...], axis=0)
    mu = cs / _N
    var = cq / _N - mu * mu
    res = (h_ref[...] - mu) / jnp.sqrt(var + 1e-5) * g_ref[...] + b_ref[...]
    out_ref[...] = jnp.concatenate(
        [res, jnp.zeros((res.shape[0], 56), jnp.float32)], axis=1)


def _bn_layer(h, colsum, colsq, p):
    nb = _N // _ROWS
    row = lambda i: (i, 0)
    return pl.pallas_call(
        _bn_body,
        grid=(nb,),
        in_specs=[
            pl.BlockSpec((_ROWS, _H), row),
            pl.BlockSpec((nb, 1, _H), lambda i: (0, 0, 0)),
            pl.BlockSpec((nb, 1, _H), lambda i: (0, 0, 0)),
            pl.BlockSpec((1, _H), lambda i: (0, 0)),
            pl.BlockSpec((1, _H), lambda i: (0, 0)),
        ],
        out_specs=pl.BlockSpec((_ROWS, _H + 56), row),
        out_shape=jax.ShapeDtypeStruct((_N, _H + 56), jnp.float32),
    )(h, colsum, colsq, p["gamma"].reshape(1, _H), p["beta"].reshape(1, _H))


def kernel(x, edge_index, params):
    src = edge_index[0]
    dst = edge_index[1]
    order = jnp.argsort(dst, stable=True)
    src_s = src[order]
    dst_s = jnp.concatenate(
        [dst[order], jnp.full((96,), _N, jnp.int32)])
    cuts = jnp.arange(_NW * _NRND + 1, dtype=jnp.int32) * _NPB
    offs = jnp.searchsorted(dst_s[:_E], cuts, side="left").astype(jnp.int32)
    offs = jnp.stack([offs[:-1], offs[1:]], axis=1)
    offs = jnp.pad(offs, ((0, 0), (0, 14)))
    x128 = jnp.pad(x, ((0, 0), (0, 125)))

    cnt = None
    h208 = None
    for i, p in enumerate(params):
        if i == 0:
            cpt, cp, table, h_in, C = 128, 16, x128, x, 3
        else:
            cpt, cp, table, h_in, C = 256, 256, h208, h208, 200
        msgs = _make_gather(cpt)(table, src_s)[0]
        s, s2, mn, mx, cnt_i = _make_accum(cpt, cp)(msgs, dst_s, offs)
        if i == 0:
            cnt = cnt_i
        out, colsum, colsq = _dense_layer(cnt, s, s2, mn, mx, h_in, p, C)
        h208 = _bn_layer(out, colsum, colsq, p)
    return h208[:, :_H]
